# baseline (speedup 1.0000x reference)
import functools

import jax
import jax.numpy as jnp
from jax import lax
from jax.experimental import pallas as pl
from jax.experimental.pallas import tpu as pltpu
from jax.experimental.pallas import tpu_sc as plsc

N = 10000
E = 320000
IN_F = 128
HID = 256

NC = 2
NS = 16
L = 16

EROWS = 2560
EPAD = EROWS * 128
ROWS_PER_TILE = EROWS // NS
ROWS_PER_WORKER = EROWS // (NC * NS)
CHUNK_ROWS = 32
PAD_SRC = 0
PAD_DST = 10008

NACC = 10112
ACC_PER_TILE = NACC // NS
HROWS = 128
HB = HROWS // NS

ROW_BLK = 2000

_sc_mesh = plsc.VectorSubcoreMesh(core_axis_name="c", subcore_axis_name="s")


HSIZE = HROWS * 128
HSEG = HSIZE // NS


def _deg_body(dst_hbm, out_hbm, idx_v, hist_v, red_v, outp_v, shared_h):
    c = lax.axis_index("c")
    s = lax.axis_index("s")
    w = c * NS + s
    pltpu.sync_copy(dst_hbm.at[pl.ds(w * ROWS_PER_WORKER, ROWS_PER_WORKER)], idx_v)
    zero16 = jnp.zeros((L,), jnp.float32)
    ones16 = jnp.ones((L,), jnp.float32)

    def zbody(i, carry):
        hist_v[pl.ds(i * L, L)] = zero16
        return carry
    lax.fori_loop(0, HSIZE // L, zbody, 0)

    def hbody(r, carry):
        for k in range(128 // L):
            idx16 = idx_v[r, pl.ds(k * L, L)]
            plsc.addupdate_scatter(hist_v, [idx16], ones16)
        return carry
    lax.fori_loop(0, ROWS_PER_WORKER, hbody, 0)

    pltpu.sync_copy(hist_v, shared_h.at[s])
    plsc.subcore_barrier()
    for t in range(NS):
        pltpu.sync_copy(shared_h.at[t].at[pl.ds(s * HSEG, HSEG)], red_v.at[t])
    for k in range(HSEG // L):
        acc = red_v[0, pl.ds(k * L, L)]
        for t in range(1, NS):
            acc = acc + red_v[t, pl.ds(k * L, L)]
        outp_v[pl.ds(k * L, L)] = acc
    pltpu.sync_copy(outp_v, out_hbm.at[c].at[pl.ds(s * HSEG, HSEG)])


@functools.partial(
    pl.kernel,
    out_type=jax.ShapeDtypeStruct((NC, HSIZE), jnp.float32),
    mesh=_sc_mesh,
    scratch_types=[
        pltpu.VMEM((ROWS_PER_WORKER, 128), jnp.int32),
        pltpu.VMEM((HSIZE,), jnp.float32),
        pltpu.VMEM((NS, HSEG), jnp.float32),
        pltpu.VMEM((HSEG,), jnp.float32),
        pltpu.VMEM_SHARED((NS, HSIZE), jnp.float32),
    ],
    compiler_params=pltpu.CompilerParams(needs_layout_passes=False),
)
def sc_degree(dst_hbm, out_hbm, idx_v, hist_v, red_v, outp_v, shared_h):
    _deg_body(dst_hbm, out_hbm, idx_v, hist_v, red_v, outp_v, shared_h)


def _scatter_body(y_hbm, src_hbm, dst_hbm, out_hbm, src_v, dst_v, rows_a,
                  rows_b, acc_sh, sem_a, sem_b):
    c = lax.axis_index("c")
    s = lax.axis_index("s")
    pltpu.sync_copy(y_hbm.at[c].at[pl.ds(s * ACC_PER_TILE, ACC_PER_TILE)],
                    acc_sh.at[pl.ds(s * ACC_PER_TILE, ACC_PER_TILE)])
    plsc.subcore_barrier()

    npairs = CHUNK_ROWS // 2

    def wait_gather(rows_v, sem):
        pltpu.make_async_copy(y_hbm.at[c].at[pl.ds(0, 128)], rows_v, sem).wait()

    def chunk_body(ch, carry):
        base = s * ROWS_PER_TILE + ch * CHUNK_ROWS
        pltpu.sync_copy(src_hbm.at[pl.ds(base, CHUNK_ROWS)], src_v)
        pltpu.sync_copy(dst_hbm.at[pl.ds(base, CHUNK_ROWS)], dst_v)
        pltpu.async_copy(y_hbm.at[c].at[src_v.at[0]], rows_a, sem_a)

        def pair_body(q, c2):
            j0 = 2 * q
            wait_gather(rows_a, sem_a)
            pltpu.async_copy(y_hbm.at[c].at[src_v.at[j0 + 1]], rows_b, sem_b)
            pltpu.sync_copy(rows_a, acc_sh.at[dst_v.at[j0]], add=True)
            wait_gather(rows_b, sem_b)

            @pl.when(q < npairs - 1)
            def _():
                pltpu.async_copy(y_hbm.at[c].at[src_v.at[j0 + 2]], rows_a,
                                 sem_a)
            pltpu.sync_copy(rows_b, acc_sh.at[dst_v.at[j0 + 1]], add=True)
            return c2
        lax.fori_loop(0, npairs, pair_body, 0)
        return carry
    lax.fori_loop(0, ROWS_PER_TILE // CHUNK_ROWS, chunk_body, 0)

    plsc.subcore_barrier()
    pltpu.sync_copy(acc_sh.at[pl.ds(s * ACC_PER_TILE, ACC_PER_TILE)],
                    out_hbm.at[c].at[pl.ds(s * ACC_PER_TILE, ACC_PER_TILE)])


@functools.partial(
    pl.kernel,
    out_type=jax.ShapeDtypeStruct((NC, NACC, 128), jnp.float32),
    mesh=_sc_mesh,
    scratch_types=[
        pltpu.VMEM((CHUNK_ROWS, 128), jnp.int32),
        pltpu.VMEM((CHUNK_ROWS, 128), jnp.int32),
        pltpu.VMEM((128, 128), jnp.float32),
        pltpu.VMEM((128, 128), jnp.float32),
        pltpu.VMEM_SHARED((NACC, 128), jnp.float32),
        pltpu.SemaphoreType.DMA,
        pltpu.SemaphoreType.DMA,
    ],
    compiler_params=pltpu.CompilerParams(needs_layout_passes=False),
)
def sc_scatter(y_hbm, src_hbm, dst_hbm, out_hbm, src_v, dst_v, rows_a, rows_b,
               acc_sh, sem_a, sem_b):
    _scatter_body(y_hbm, src_hbm, dst_hbm, out_hbm, src_v, dst_v, rows_a,
                  rows_b, acc_sh, sem_a, sem_b)


def _mm1_body(x_ref, w_ref, d0_ref, d1_ref, o_ref):
    dis = lax.rsqrt(d0_ref[...] + d1_ref[...] + 1.0)
    xw = jnp.dot(x_ref[...], w_ref[...], preferred_element_type=jnp.float32)
    o_ref[0] = xw * dis


def tc_mm1(x, W1, deg0, deg1):
    return pl.pallas_call(
        _mm1_body,
        grid=(N // ROW_BLK, HID // 128),
        in_specs=[
            pl.BlockSpec((ROW_BLK, IN_F), lambda i, j: (i, 0)),
            pl.BlockSpec((IN_F, 128), lambda i, j: (0, j)),
            pl.BlockSpec((ROW_BLK, 1), lambda i, j: (i, 0)),
            pl.BlockSpec((ROW_BLK, 1), lambda i, j: (i, 0)),
        ],
        out_specs=pl.BlockSpec((1, ROW_BLK, 128), lambda i, j: (j, i, 0)),
        out_shape=jax.ShapeDtypeStruct((NC, NACC, 128), jnp.float32),
    )(x, W1, deg0, deg1)


def _mm2_body(a_ref, w_ref, d0_ref, d1_ref, b_ref, o_ref):
    dis = lax.rsqrt(d0_ref[...] + d1_ref[...] + 1.0)
    h = jnp.concatenate([a_ref[0], a_ref[1]], axis=1)
    h1 = jnp.maximum(h * dis + b_ref[...], 0.0)
    o_ref[0] = jnp.dot(h1, w_ref[...], preferred_element_type=jnp.float32) * dis


def tc_mm2(acc, W2, deg0, deg1, b1):
    return pl.pallas_call(
        _mm2_body,
        grid=(N // ROW_BLK, HID // 128),
        in_specs=[
            pl.BlockSpec((NC, ROW_BLK, 128), lambda i, j: (0, i, 0)),
            pl.BlockSpec((HID, 128), lambda i, j: (0, j)),
            pl.BlockSpec((ROW_BLK, 1), lambda i, j: (i, 0)),
            pl.BlockSpec((ROW_BLK, 1), lambda i, j: (i, 0)),
            pl.BlockSpec((1, HID), lambda i, j: (0, 0)),
        ],
        out_specs=pl.BlockSpec((1, ROW_BLK, 128), lambda i, j: (j, i, 0)),
        out_shape=jax.ShapeDtypeStruct((NC, NACC, 128), jnp.float32),
    )(acc, W2, deg0, deg1, b1)


def _heads_body(a_ref, d0_ref, d1_ref, b2_ref, dw1_ref, db1_ref, dw2_ref,
                db2_ref, cw1_ref, cb1_ref, cw2_ref, cb2_ref, ob_ref, ot_ref):
    dis = lax.rsqrt(d0_ref[...] + d1_ref[...] + 1.0)
    h = jnp.concatenate([a_ref[0], a_ref[1]], axis=1)
    h2 = jnp.maximum(h * dis + b2_ref[...], 0.0)
    t1 = jnp.maximum(
        jnp.dot(h2, dw1_ref[...], preferred_element_type=jnp.float32)
        + db1_ref[...], 0.0)
    ob_ref[...] = (jnp.dot(t1, dw2_ref[...], preferred_element_type=jnp.float32)
                   + db2_ref[...])
    t2 = jnp.maximum(
        jnp.dot(h2, cw1_ref[...], preferred_element_type=jnp.float32)
        + cb1_ref[...], 0.0)
    ot_ref[...] = (jnp.dot(t2, cw2_ref[...], preferred_element_type=jnp.float32)
                   + cb2_ref[...])


def tc_heads(acc, deg0, deg1, b2, dW1c, db1, dW2p, db2p, cW1c, cb1, cW2p, cb2p):
    return pl.pallas_call(
        _heads_body,
        grid=(N // ROW_BLK,),
        in_specs=[
            pl.BlockSpec((NC, ROW_BLK, 128), lambda i: (0, i, 0)),
            pl.BlockSpec((ROW_BLK, 1), lambda i: (i, 0)),
            pl.BlockSpec((ROW_BLK, 1), lambda i: (i, 0)),
            pl.BlockSpec((1, HID), lambda i: (0, 0)),
            pl.BlockSpec((HID, HID), lambda i: (0, 0)),
            pl.BlockSpec((1, HID), lambda i: (0, 0)),
            pl.BlockSpec((HID, 128), lambda i: (0, 0)),
            pl.BlockSpec((1, 128), lambda i: (0, 0)),
            pl.BlockSpec((HID, HID), lambda i: (0, 0)),
            pl.BlockSpec((1, HID), lambda i: (0, 0)),
            pl.BlockSpec((HID, 128), lambda i: (0, 0)),
            pl.BlockSpec((1, 128), lambda i: (0, 0)),
        ],
        out_specs=[
            pl.BlockSpec((ROW_BLK, 128), lambda i: (i, 0)),
            pl.BlockSpec((ROW_BLK, 128), lambda i: (i, 0)),
        ],
        out_shape=[
            jax.ShapeDtypeStruct((N, 128), jnp.float32),
            jax.ShapeDtypeStruct((N, 128), jnp.float32),
        ],
    )(acc, deg0, deg1, b2, dW1c, db1, dW2p, db2p, cW1c, cb1, cW2p, cb2p)


def kernel(x, edge_index, W1, b1, W2, b2, dW1, db1, dW2, db2, cW1, cb1, cW2,
           cb2):
    src = edge_index[0]
    dst = edge_index[1]
    pad = EPAD - E
    padv = jnp.arange(pad, dtype=jnp.int32)
    src2d = jnp.concatenate([src, padv % N]).reshape(EROWS, 128)
    dst2d = jnp.concatenate([dst, N + padv % (NACC - N)]).reshape(EROWS, 128)

    degp = sc_degree(dst2d)
    deg0 = degp[0][:N][:, None]
    deg1 = degp[1][:N][:, None]

    y1 = tc_mm1(x, W1, deg0, deg1)
    acc1 = sc_scatter(y1, src2d, dst2d)
    y2 = tc_mm2(acc1, W2, deg0, deg1, b1.reshape(1, HID))
    acc2 = sc_scatter(y2, src2d, dst2d)

    dW2p = jnp.pad(dW2, ((0, 0), (0, 128 - dW2.shape[1])))
    db2p = jnp.pad(db2, (0, 128 - db2.shape[0])).reshape(1, 128)
    cW2p = jnp.pad(cW2, ((0, 0), (0, 128 - cW2.shape[1])))
    cb2p = jnp.pad(cb2, (0, 128 - cb2.shape[0])).reshape(1, 128)
    outb, outt = tc_heads(acc2, deg0, deg1, b2.reshape(1, HID), dW1[:HID],
                          db1.reshape(1, HID), dW2p, db2p, cW1[:HID],
                          cb1.reshape(1, HID), cW2p, cb2p)
    return outb[:, :dW2.shape[1]], outt[:, :cW2.shape[1]]


# speedup vs baseline: 1.1633x; 1.1633x over previous
import functools

import jax
import jax.numpy as jnp
from jax import lax
from jax.experimental import pallas as pl
from jax.experimental.pallas import tpu as pltpu
from jax.experimental.pallas import tpu_sc as plsc

N = 10000
E = 320000
IN_F = 128
HID = 256

NC = 2
NS = 16
L = 16

EROWS = 2560
EPAD = EROWS * 128
ROWS_PER_TILE = EROWS // NS
ROWS_PER_WORKER = EROWS // (NC * NS)
CHUNK_ROWS = 32
PAD_SRC = 0
PAD_DST = 10008

NACC = 10112
ACC_PER_TILE = NACC // NS
HROWS = 128
HB = HROWS // NS

ROW_BLK = 2000

_sc_mesh = plsc.VectorSubcoreMesh(core_axis_name="c", subcore_axis_name="s")


HSIZE = HROWS * 128
HSEG = HSIZE // NS


def _deg_body(dst_hbm, out_hbm, idx_v, hist_v, red_v, outp_v, shared_h):
    c = lax.axis_index("c")
    s = lax.axis_index("s")
    w = c * NS + s
    pltpu.sync_copy(dst_hbm.at[pl.ds(w * ROWS_PER_WORKER, ROWS_PER_WORKER)], idx_v)
    zero16 = jnp.zeros((L,), jnp.float32)
    ones16 = jnp.ones((L,), jnp.float32)

    def zbody(i, carry):
        hist_v[pl.ds(i * L, L)] = zero16
        return carry
    lax.fori_loop(0, HSIZE // L, zbody, 0)

    def hbody(r, carry):
        for k in range(128 // L):
            idx16 = idx_v[r, pl.ds(k * L, L)]
            plsc.addupdate_scatter(hist_v, [idx16], ones16)
        return carry
    lax.fori_loop(0, ROWS_PER_WORKER, hbody, 0)

    pltpu.sync_copy(hist_v, shared_h.at[s])
    plsc.subcore_barrier()
    for t in range(NS):
        pltpu.sync_copy(shared_h.at[t].at[pl.ds(s * HSEG, HSEG)], red_v.at[t])
    for k in range(HSEG // L):
        acc = red_v[0, pl.ds(k * L, L)]
        for t in range(1, NS):
            acc = acc + red_v[t, pl.ds(k * L, L)]
        outp_v[pl.ds(k * L, L)] = acc
    pltpu.sync_copy(outp_v, out_hbm.at[c].at[pl.ds(s * HSEG, HSEG)])


@functools.partial(
    pl.kernel,
    out_type=jax.ShapeDtypeStruct((NC, HSIZE), jnp.float32),
    mesh=_sc_mesh,
    scratch_types=[
        pltpu.VMEM((ROWS_PER_WORKER, 128), jnp.int32),
        pltpu.VMEM((HSIZE,), jnp.float32),
        pltpu.VMEM((NS, HSEG), jnp.float32),
        pltpu.VMEM((HSEG,), jnp.float32),
        pltpu.VMEM_SHARED((NS, HSIZE), jnp.float32),
    ],
    compiler_params=pltpu.CompilerParams(needs_layout_passes=False),
)
def sc_degree(dst_hbm, out_hbm, idx_v, hist_v, red_v, outp_v, shared_h):
    _deg_body(dst_hbm, out_hbm, idx_v, hist_v, red_v, outp_v, shared_h)


def _scatter_body(y_hbm, src_hbm, dst_hbm, out_hbm, src_v, dst_v, rows_a,
                  rows_b, acc_sh, sem_a, sem_b):
    c = lax.axis_index("c")
    s = lax.axis_index("s")
    pltpu.sync_copy(y_hbm.at[c].at[pl.ds(s * ACC_PER_TILE, ACC_PER_TILE)],
                    acc_sh.at[pl.ds(s * ACC_PER_TILE, ACC_PER_TILE)])
    plsc.subcore_barrier()

    npairs = CHUNK_ROWS // 2

    def wait_gather(rows_v, sem):
        pltpu.make_async_copy(y_hbm.at[c].at[pl.ds(0, 128)], rows_v, sem).wait()

    def chunk_body(ch, carry):
        base = s * ROWS_PER_TILE + ch * CHUNK_ROWS
        pltpu.sync_copy(src_hbm.at[pl.ds(base, CHUNK_ROWS)], src_v)
        pltpu.sync_copy(dst_hbm.at[pl.ds(base, CHUNK_ROWS)], dst_v)
        pltpu.async_copy(y_hbm.at[c].at[src_v.at[0]], rows_a, sem_a)
        pltpu.async_copy(y_hbm.at[c].at[src_v.at[1]], rows_b, sem_b)

        def pair_body(q, c2):
            j0 = 2 * q
            wait_gather(rows_a, sem_a)
            pltpu.sync_copy(rows_a, acc_sh.at[dst_v.at[j0]], add=True)

            @pl.when(q < npairs - 1)
            def _():
                pltpu.async_copy(y_hbm.at[c].at[src_v.at[j0 + 2]], rows_a,
                                 sem_a)
            wait_gather(rows_b, sem_b)
            pltpu.sync_copy(rows_b, acc_sh.at[dst_v.at[j0 + 1]], add=True)

            @pl.when(q < npairs - 1)
            def _():
                pltpu.async_copy(y_hbm.at[c].at[src_v.at[j0 + 3]], rows_b,
                                 sem_b)
            return c2
        lax.fori_loop(0, npairs, pair_body, 0)
        return carry
    lax.fori_loop(0, ROWS_PER_TILE // CHUNK_ROWS, chunk_body, 0)

    plsc.subcore_barrier()
    pltpu.sync_copy(acc_sh.at[pl.ds(s * ACC_PER_TILE, ACC_PER_TILE)],
                    out_hbm.at[c].at[pl.ds(s * ACC_PER_TILE, ACC_PER_TILE)])


@functools.partial(
    pl.kernel,
    out_type=jax.ShapeDtypeStruct((NC, NACC, 128), jnp.float32),
    mesh=_sc_mesh,
    scratch_types=[
        pltpu.VMEM((CHUNK_ROWS, 128), jnp.int32),
        pltpu.VMEM((CHUNK_ROWS, 128), jnp.int32),
        pltpu.VMEM((128, 128), jnp.float32),
        pltpu.VMEM((128, 128), jnp.float32),
        pltpu.VMEM_SHARED((NACC, 128), jnp.float32),
        pltpu.SemaphoreType.DMA,
        pltpu.SemaphoreType.DMA,
    ],
    compiler_params=pltpu.CompilerParams(needs_layout_passes=False),
)
def sc_scatter(y_hbm, src_hbm, dst_hbm, out_hbm, src_v, dst_v, rows_a, rows_b,
               acc_sh, sem_a, sem_b):
    _scatter_body(y_hbm, src_hbm, dst_hbm, out_hbm, src_v, dst_v, rows_a,
                  rows_b, acc_sh, sem_a, sem_b)


def _mm1_body(x_ref, w_ref, d0_ref, d1_ref, o_ref):
    dis = lax.rsqrt(d0_ref[...] + d1_ref[...] + 1.0)
    xw = jnp.dot(x_ref[...], w_ref[...], preferred_element_type=jnp.float32)
    o_ref[0] = xw * dis


def tc_mm1(x, W1, deg0, deg1):
    return pl.pallas_call(
        _mm1_body,
        grid=(N // ROW_BLK, HID // 128),
        in_specs=[
            pl.BlockSpec((ROW_BLK, IN_F), lambda i, j: (i, 0)),
            pl.BlockSpec((IN_F, 128), lambda i, j: (0, j)),
            pl.BlockSpec((ROW_BLK, 1), lambda i, j: (i, 0)),
            pl.BlockSpec((ROW_BLK, 1), lambda i, j: (i, 0)),
        ],
        out_specs=pl.BlockSpec((1, ROW_BLK, 128), lambda i, j: (j, i, 0)),
        out_shape=jax.ShapeDtypeStruct((NC, NACC, 128), jnp.float32),
    )(x, W1, deg0, deg1)


def _mm2_body(a_ref, w_ref, d0_ref, d1_ref, b_ref, o_ref):
    dis = lax.rsqrt(d0_ref[...] + d1_ref[...] + 1.0)
    h = jnp.concatenate([a_ref[0], a_ref[1]], axis=1)
    h1 = jnp.maximum(h * dis + b_ref[...], 0.0)
    o_ref[0] = jnp.dot(h1, w_ref[...], preferred_element_type=jnp.float32) * dis


def tc_mm2(acc, W2, deg0, deg1, b1):
    return pl.pallas_call(
        _mm2_body,
        grid=(N // ROW_BLK, HID // 128),
        in_specs=[
            pl.BlockSpec((NC, ROW_BLK, 128), lambda i, j: (0, i, 0)),
            pl.BlockSpec((HID, 128), lambda i, j: (0, j)),
            pl.BlockSpec((ROW_BLK, 1), lambda i, j: (i, 0)),
            pl.BlockSpec((ROW_BLK, 1), lambda i, j: (i, 0)),
            pl.BlockSpec((1, HID), lambda i, j: (0, 0)),
        ],
        out_specs=pl.BlockSpec((1, ROW_BLK, 128), lambda i, j: (j, i, 0)),
        out_shape=jax.ShapeDtypeStruct((NC, NACC, 128), jnp.float32),
    )(acc, W2, deg0, deg1, b1)


def _heads_body(a_ref, d0_ref, d1_ref, b2_ref, dw1_ref, db1_ref, dw2_ref,
                db2_ref, cw1_ref, cb1_ref, cw2_ref, cb2_ref, ob_ref, ot_ref):
    dis = lax.rsqrt(d0_ref[...] + d1_ref[...] + 1.0)
    h = jnp.concatenate([a_ref[0], a_ref[1]], axis=1)
    h2 = jnp.maximum(h * dis + b2_ref[...], 0.0)
    t1 = jnp.maximum(
        jnp.dot(h2, dw1_ref[...], preferred_element_type=jnp.float32)
        + db1_ref[...], 0.0)
    ob_ref[...] = (jnp.dot(t1, dw2_ref[...], preferred_element_type=jnp.float32)
                   + db2_ref[...])
    t2 = jnp.maximum(
        jnp.dot(h2, cw1_ref[...], preferred_element_type=jnp.float32)
        + cb1_ref[...], 0.0)
    ot_ref[...] = (jnp.dot(t2, cw2_ref[...], preferred_element_type=jnp.float32)
                   + cb2_ref[...])


def tc_heads(acc, deg0, deg1, b2, dW1c, db1, dW2p, db2p, cW1c, cb1, cW2p, cb2p):
    return pl.pallas_call(
        _heads_body,
        grid=(N // ROW_BLK,),
        in_specs=[
            pl.BlockSpec((NC, ROW_BLK, 128), lambda i: (0, i, 0)),
            pl.BlockSpec((ROW_BLK, 1), lambda i: (i, 0)),
            pl.BlockSpec((ROW_BLK, 1), lambda i: (i, 0)),
            pl.BlockSpec((1, HID), lambda i: (0, 0)),
            pl.BlockSpec((HID, HID), lambda i: (0, 0)),
            pl.BlockSpec((1, HID), lambda i: (0, 0)),
            pl.BlockSpec((HID, 128), lambda i: (0, 0)),
            pl.BlockSpec((1, 128), lambda i: (0, 0)),
            pl.BlockSpec((HID, HID), lambda i: (0, 0)),
            pl.BlockSpec((1, HID), lambda i: (0, 0)),
            pl.BlockSpec((HID, 128), lambda i: (0, 0)),
            pl.BlockSpec((1, 128), lambda i: (0, 0)),
        ],
        out_specs=[
            pl.BlockSpec((ROW_BLK, 128), lambda i: (i, 0)),
            pl.BlockSpec((ROW_BLK, 128), lambda i: (i, 0)),
        ],
        out_shape=[
            jax.ShapeDtypeStruct((N, 128), jnp.float32),
            jax.ShapeDtypeStruct((N, 128), jnp.float32),
        ],
    )(acc, deg0, deg1, b2, dW1c, db1, dW2p, db2p, cW1c, cb1, cW2p, cb2p)


def kernel(x, edge_index, W1, b1, W2, b2, dW1, db1, dW2, db2, cW1, cb1, cW2,
           cb2):
    src = edge_index[0]
    dst = edge_index[1]
    pad = EPAD - E
    padv = jnp.arange(pad, dtype=jnp.int32)
    src2d = jnp.concatenate([src, padv % N]).reshape(EROWS, 128)
    dst2d = jnp.concatenate([dst, N + padv % (NACC - N)]).reshape(EROWS, 128)

    degp = sc_degree(dst2d)
    deg0 = degp[0][:N][:, None]
    deg1 = degp[1][:N][:, None]

    y1 = tc_mm1(x, W1, deg0, deg1)
    acc1 = sc_scatter(y1, src2d, dst2d)
    y2 = tc_mm2(acc1, W2, deg0, deg1, b1.reshape(1, HID))
    acc2 = sc_scatter(y2, src2d, dst2d)

    dW2p = jnp.pad(dW2, ((0, 0), (0, 128 - dW2.shape[1])))
    db2p = jnp.pad(db2, (0, 128 - db2.shape[0])).reshape(1, 128)
    cW2p = jnp.pad(cW2, ((0, 0), (0, 128 - cW2.shape[1])))
    cb2p = jnp.pad(cb2, (0, 128 - cb2.shape[0])).reshape(1, 128)
    outb, outt = tc_heads(acc2, deg0, deg1, b2.reshape(1, HID), dW1[:HID],
                          db1.reshape(1, HID), dW2p, db2p, cW1[:HID],
                          cb1.reshape(1, HID), cW2p, cb2p)
    return outb[:, :dW2.shape[1]], outt[:, :cW2.shape[1]]


# speedup vs baseline: 1.1749x; 1.0099x over previous
import functools

import jax
import jax.numpy as jnp
from jax import lax
from jax.experimental import pallas as pl
from jax.experimental.pallas import tpu as pltpu
from jax.experimental.pallas import tpu_sc as plsc

N = 10000
E = 320000
IN_F = 128
HID = 256
NUM_TYPES = 8

NC = 2
NS = 16
L = 16

EROWS = 2560
EPAD = EROWS * 128
ROWS_PER_TILE = EROWS // NS
ROWS_PER_WORKER = EROWS // (NC * NS)
CHUNK_ROWS = 40
PAD_SRC = 0
PAD_DST = 10008

NACC = 10112
ACC_PER_TILE = NACC // NS
HROWS = 128
HB = HROWS // NS

ROW_BLK = 2000

_sc_mesh = plsc.VectorSubcoreMesh(core_axis_name="c", subcore_axis_name="s")


HSIZE = HROWS * 128
HSEG = HSIZE // NS


def _deg_body(dst_hbm, out_hbm, idx_v, hist_v, red_v, outp_v, shared_h):
    c = lax.axis_index("c")
    s = lax.axis_index("s")
    w = c * NS + s
    pltpu.sync_copy(dst_hbm.at[pl.ds(w * ROWS_PER_WORKER, ROWS_PER_WORKER)], idx_v)
    zero16 = jnp.zeros((L,), jnp.float32)
    ones16 = jnp.ones((L,), jnp.float32)

    def zbody(i, carry):
        hist_v[pl.ds(i * L, L)] = zero16
        return carry
    lax.fori_loop(0, HSIZE // L, zbody, 0)

    def hbody(r, carry):
        for k in range(128 // L):
            idx16 = idx_v[r, pl.ds(k * L, L)]
            plsc.addupdate_scatter(hist_v, [idx16], ones16)
        return carry
    lax.fori_loop(0, ROWS_PER_WORKER, hbody, 0)

    pltpu.sync_copy(hist_v, shared_h.at[s])
    plsc.subcore_barrier()
    for t in range(NS):
        pltpu.sync_copy(shared_h.at[t].at[pl.ds(s * HSEG, HSEG)], red_v.at[t])
    for k in range(HSEG // L):
        acc = red_v[0, pl.ds(k * L, L)]
        for t in range(1, NS):
            acc = acc + red_v[t, pl.ds(k * L, L)]
        outp_v[pl.ds(k * L, L)] = acc
    pltpu.sync_copy(outp_v, out_hbm.at[c].at[pl.ds(s * HSEG, HSEG)])


@functools.partial(
    pl.kernel,
    out_type=jax.ShapeDtypeStruct((NC, HSIZE), jnp.float32),
    mesh=_sc_mesh,
    scratch_types=[
        pltpu.VMEM((ROWS_PER_WORKER, 128), jnp.int32),
        pltpu.VMEM((HSIZE,), jnp.float32),
        pltpu.VMEM((NS, HSEG), jnp.float32),
        pltpu.VMEM((HSEG,), jnp.float32),
        pltpu.VMEM_SHARED((NS, HSIZE), jnp.float32),
    ],
    compiler_params=pltpu.CompilerParams(needs_layout_passes=False),
)
def sc_degree(dst_hbm, out_hbm, idx_v, hist_v, red_v, outp_v, shared_h):
    _deg_body(dst_hbm, out_hbm, idx_v, hist_v, red_v, outp_v, shared_h)


def _scatter_body(y_hbm, src_hbm, dst_hbm, out_hbm, src_v, dst_v, rows_a,
                  rows_b, acc_sh, sem_a, sem_b):
    c = lax.axis_index("c")
    s = lax.axis_index("s")
    pltpu.sync_copy(y_hbm.at[c].at[pl.ds(s * ACC_PER_TILE, ACC_PER_TILE)],
                    acc_sh.at[pl.ds(s * ACC_PER_TILE, ACC_PER_TILE)])
    plsc.subcore_barrier()

    npairs = CHUNK_ROWS // 2

    def wait_gather(rows_v, sem):
        pltpu.make_async_copy(y_hbm.at[c].at[pl.ds(0, 128)], rows_v, sem).wait()

    def chunk_body(ch, carry):
        base = s * ROWS_PER_TILE + ch * CHUNK_ROWS
        pltpu.sync_copy(src_hbm.at[pl.ds(base, CHUNK_ROWS)], src_v)
        pltpu.sync_copy(dst_hbm.at[pl.ds(base, CHUNK_ROWS)], dst_v)
        pltpu.async_copy(y_hbm.at[c].at[src_v.at[0]], rows_a, sem_a)
        pltpu.async_copy(y_hbm.at[c].at[src_v.at[1]], rows_b, sem_b)

        def pair_body(q, c2):
            j0 = 2 * q
            wait_gather(rows_a, sem_a)
            pltpu.sync_copy(rows_a, acc_sh.at[dst_v.at[j0]], add=True)

            @pl.when(q < npairs - 1)
            def _():
                pltpu.async_copy(y_hbm.at[c].at[src_v.at[j0 + 2]], rows_a,
                                 sem_a)
            wait_gather(rows_b, sem_b)
            pltpu.sync_copy(rows_b, acc_sh.at[dst_v.at[j0 + 1]], add=True)

            @pl.when(q < npairs - 1)
            def _():
                pltpu.async_copy(y_hbm.at[c].at[src_v.at[j0 + 3]], rows_b,
                                 sem_b)
            return c2
        lax.fori_loop(0, npairs, pair_body, 0)
        return carry
    lax.fori_loop(0, ROWS_PER_TILE // CHUNK_ROWS, chunk_body, 0)

    plsc.subcore_barrier()
    pltpu.sync_copy(acc_sh.at[pl.ds(s * ACC_PER_TILE, ACC_PER_TILE)],
                    out_hbm.at[c].at[pl.ds(s * ACC_PER_TILE, ACC_PER_TILE)])


@functools.partial(
    pl.kernel,
    out_type=jax.ShapeDtypeStruct((NC, NACC, 128), jnp.float32),
    mesh=_sc_mesh,
    scratch_types=[
        pltpu.VMEM((CHUNK_ROWS, 128), jnp.int32),
        pltpu.VMEM((CHUNK_ROWS, 128), jnp.int32),
        pltpu.VMEM((128, 128), jnp.float32),
        pltpu.VMEM((128, 128), jnp.float32),
        pltpu.VMEM_SHARED((NACC, 128), jnp.float32),
        pltpu.SemaphoreType.DMA,
        pltpu.SemaphoreType.DMA,
    ],
    compiler_params=pltpu.CompilerParams(needs_layout_passes=False),
)
def sc_scatter(y_hbm, src_hbm, dst_hbm, out_hbm, src_v, dst_v, rows_a, rows_b,
               acc_sh, sem_a, sem_b):
    _scatter_body(y_hbm, src_hbm, dst_hbm, out_hbm, src_v, dst_v, rows_a,
                  rows_b, acc_sh, sem_a, sem_b)


def _mm1_body(x_ref, w_ref, d0_ref, d1_ref, o_ref):
    dis = lax.rsqrt(d0_ref[...] + d1_ref[...] + 1.0)
    xw = jnp.dot(x_ref[...], w_ref[...], preferred_element_type=jnp.float32)
    o_ref[0] = xw * dis


def tc_mm1(x, W1, deg0, deg1):
    return pl.pallas_call(
        _mm1_body,
        grid=(N // ROW_BLK, HID // 128),
        in_specs=[
            pl.BlockSpec((ROW_BLK, IN_F), lambda i, j: (i, 0)),
            pl.BlockSpec((IN_F, 128), lambda i, j: (0, j)),
            pl.BlockSpec((ROW_BLK, 1), lambda i, j: (i, 0)),
            pl.BlockSpec((ROW_BLK, 1), lambda i, j: (i, 0)),
        ],
        out_specs=pl.BlockSpec((1, ROW_BLK, 128), lambda i, j: (j, i, 0)),
        out_shape=jax.ShapeDtypeStruct((NC, NACC, 128), jnp.float32),
    )(x, W1, deg0, deg1)


def _mm2_body(a_ref, w_ref, d0_ref, d1_ref, b_ref, o_ref):
    dis = lax.rsqrt(d0_ref[...] + d1_ref[...] + 1.0)
    h = jnp.concatenate([a_ref[0], a_ref[1]], axis=1)
    h1 = jnp.maximum(h * dis + b_ref[...], 0.0)
    o_ref[0] = jnp.dot(h1, w_ref[...], preferred_element_type=jnp.float32) * dis


def tc_mm2(acc, W2, deg0, deg1, b1):
    return pl.pallas_call(
        _mm2_body,
        grid=(N // ROW_BLK, HID // 128),
        in_specs=[
            pl.BlockSpec((NC, ROW_BLK, 128), lambda i, j: (0, i, 0)),
            pl.BlockSpec((HID, 128), lambda i, j: (0, j)),
            pl.BlockSpec((ROW_BLK, 1), lambda i, j: (i, 0)),
            pl.BlockSpec((ROW_BLK, 1), lambda i, j: (i, 0)),
            pl.BlockSpec((1, HID), lambda i, j: (0, 0)),
        ],
        out_specs=pl.BlockSpec((1, ROW_BLK, 128), lambda i, j: (j, i, 0)),
        out_shape=jax.ShapeDtypeStruct((NC, NACC, 128), jnp.float32),
    )(acc, W2, deg0, deg1, b1)


def _heads_body(a_ref, d0_ref, d1_ref, b2_ref, dw1_ref, db1_ref, dw2_ref,
                db2_ref, cw1_ref, cb1_ref, cw2_ref, cb2_ref, ob_ref, ot_ref):
    dis = lax.rsqrt(d0_ref[...] + d1_ref[...] + 1.0)
    h = jnp.concatenate([a_ref[0], a_ref[1]], axis=1)
    h2 = jnp.maximum(h * dis + b2_ref[...], 0.0)
    t1 = jnp.maximum(
        jnp.dot(h2, dw1_ref[...], preferred_element_type=jnp.float32)
        + db1_ref[...], 0.0)
    ob_ref[...] = (jnp.dot(t1, dw2_ref[...], preferred_element_type=jnp.float32)
                   + db2_ref[...])
    t2 = jnp.maximum(
        jnp.dot(h2, cw1_ref[...], preferred_element_type=jnp.float32)
        + cb1_ref[...], 0.0)
    ot_ref[...] = (jnp.dot(t2, cw2_ref[...], preferred_element_type=jnp.float32)
                   + cb2_ref[...])


def tc_heads(acc, deg0, deg1, b2, dW1c, db1, dW2p, db2p, cW1c, cb1, cW2p, cb2p):
    return pl.pallas_call(
        _heads_body,
        grid=(N // ROW_BLK,),
        in_specs=[
            pl.BlockSpec((NC, ROW_BLK, 128), lambda i: (0, i, 0)),
            pl.BlockSpec((ROW_BLK, 1), lambda i: (i, 0)),
            pl.BlockSpec((ROW_BLK, 1), lambda i: (i, 0)),
            pl.BlockSpec((1, HID), lambda i: (0, 0)),
            pl.BlockSpec((HID, HID), lambda i: (0, 0)),
            pl.BlockSpec((1, HID), lambda i: (0, 0)),
            pl.BlockSpec((HID, 2), lambda i: (0, 0)),
            pl.BlockSpec((1, 2), lambda i: (0, 0)),
            pl.BlockSpec((HID, HID), lambda i: (0, 0)),
            pl.BlockSpec((1, HID), lambda i: (0, 0)),
            pl.BlockSpec((HID, 8), lambda i: (0, 0)),
            pl.BlockSpec((1, 8), lambda i: (0, 0)),
        ],
        out_specs=[
            pl.BlockSpec((ROW_BLK, 2), lambda i: (i, 0)),
            pl.BlockSpec((ROW_BLK, 8), lambda i: (i, 0)),
        ],
        out_shape=[
            jax.ShapeDtypeStruct((N, 2), jnp.float32),
            jax.ShapeDtypeStruct((N, 8), jnp.float32),
        ],
    )(acc, deg0, deg1, b2, dW1c, db1, dW2p, db2p, cW1c, cb1, cW2p, cb2p)


def kernel(x, edge_index, W1, b1, W2, b2, dW1, db1, dW2, db2, cW1, cb1, cW2,
           cb2):
    src = edge_index[0]
    dst = edge_index[1]
    pad = EPAD - E
    padv = jnp.arange(pad, dtype=jnp.int32)
    src2d = jnp.concatenate([src, padv % N]).reshape(EROWS, 128)
    dst2d = jnp.concatenate([dst, N + padv % (NACC - N)]).reshape(EROWS, 128)

    degp = sc_degree(dst2d)
    deg0 = degp[0][:N][:, None]
    deg1 = degp[1][:N][:, None]

    y1 = tc_mm1(x, W1, deg0, deg1)
    acc1 = sc_scatter(y1, src2d, dst2d)
    y2 = tc_mm2(acc1, W2, deg0, deg1, b1.reshape(1, HID))
    acc2 = sc_scatter(y2, src2d, dst2d)

    outb, outt = tc_heads(acc2, deg0, deg1, b2.reshape(1, HID), dW1[:HID],
                          db1.reshape(1, HID), dW2, db2.reshape(1, 2),
                          cW1[:HID], cb1.reshape(1, HID), cW2,
                          cb2.reshape(1, NUM_TYPES))
    return outb, outt


# speedup vs baseline: 1.1876x; 1.0108x over previous
import functools

import jax
import jax.numpy as jnp
from jax import lax
from jax.experimental import pallas as pl
from jax.experimental.pallas import tpu as pltpu
from jax.experimental.pallas import tpu_sc as plsc

N = 10000
E = 320000
IN_F = 128
HID = 256
NUM_TYPES = 8

NC = 2
NS = 16
L = 16

EROWS = 2560
EPAD = EROWS * 128
ROWS_PER_TILE = EROWS // NS
ROWS_PER_WORKER = EROWS // (NC * NS)
CHUNK_ROWS = 32
PAD_SRC = 0
PAD_DST = 10008

NACC = 10112
ACC_PER_TILE = NACC // NS
HROWS = 128
HB = HROWS // NS

ROW_BLK = 2000

_sc_mesh = plsc.VectorSubcoreMesh(core_axis_name="c", subcore_axis_name="s")


HSIZE = HROWS * 128
HSEG = HSIZE // NS


def _deg_body(dst_hbm, out_hbm, idx_v, hist_v, red_v, outp_v, shared_h):
    c = lax.axis_index("c")
    s = lax.axis_index("s")
    w = c * NS + s
    pltpu.sync_copy(dst_hbm.at[pl.ds(w * ROWS_PER_WORKER, ROWS_PER_WORKER)], idx_v)
    zero16 = jnp.zeros((L,), jnp.float32)
    ones16 = jnp.ones((L,), jnp.float32)

    def zbody(i, carry):
        hist_v[pl.ds(i * L, L)] = zero16
        return carry
    lax.fori_loop(0, HSIZE // L, zbody, 0)

    def hbody(r, carry):
        for k in range(128 // L):
            idx16 = idx_v[r, pl.ds(k * L, L)]
            plsc.addupdate_scatter(hist_v, [idx16], ones16)
        return carry
    lax.fori_loop(0, ROWS_PER_WORKER, hbody, 0)

    pltpu.sync_copy(hist_v, shared_h.at[s])
    plsc.subcore_barrier()
    for t in range(NS):
        pltpu.sync_copy(shared_h.at[t].at[pl.ds(s * HSEG, HSEG)], red_v.at[t])
    for k in range(HSEG // L):
        acc = red_v[0, pl.ds(k * L, L)]
        for t in range(1, NS):
            acc = acc + red_v[t, pl.ds(k * L, L)]
        outp_v[pl.ds(k * L, L)] = acc
    pltpu.sync_copy(outp_v, out_hbm.at[c].at[pl.ds(s * HSEG, HSEG)])


@functools.partial(
    pl.kernel,
    out_type=jax.ShapeDtypeStruct((NC, HSIZE), jnp.float32),
    mesh=_sc_mesh,
    scratch_types=[
        pltpu.VMEM((ROWS_PER_WORKER, 128), jnp.int32),
        pltpu.VMEM((HSIZE,), jnp.float32),
        pltpu.VMEM((NS, HSEG), jnp.float32),
        pltpu.VMEM((HSEG,), jnp.float32),
        pltpu.VMEM_SHARED((NS, HSIZE), jnp.float32),
    ],
    compiler_params=pltpu.CompilerParams(needs_layout_passes=False),
)
def sc_degree(dst_hbm, out_hbm, idx_v, hist_v, red_v, outp_v, shared_h):
    _deg_body(dst_hbm, out_hbm, idx_v, hist_v, red_v, outp_v, shared_h)


NCHUNKS = ROWS_PER_TILE // CHUNK_ROWS


def _scatter_body(y_hbm, src_hbm, dst_hbm, out_hbm, src_a, dst_a, src_b,
                  dst_b, rows_a, rows_b, acc_sh, sem_a, sem_b, sem_ia, sem_ib,
                  sem_init):
    c = lax.axis_index("c")
    s = lax.axis_index("s")
    rowsl = pl.ds(s * ACC_PER_TILE, ACC_PER_TILE)
    npairs = CHUNK_ROWS // 2

    init_d = pltpu.async_copy(y_hbm.at[c].at[rowsl], acc_sh.at[rowsl],
                              sem_init)
    base0 = s * ROWS_PER_TILE
    pltpu.async_copy(src_hbm.at[pl.ds(base0, CHUNK_ROWS)], src_a, sem_ia)
    pltpu.async_copy(dst_hbm.at[pl.ds(base0, CHUNK_ROWS)], dst_a, sem_ia)

    def wait_gather(rows_v, sem):
        pltpu.make_async_copy(y_hbm.at[c].at[pl.ds(0, 128)], rows_v, sem).wait()

    def wait_idx(src_v, dst_v, sem):
        pltpu.make_async_copy(src_hbm.at[pl.ds(0, CHUNK_ROWS)], src_v,
                              sem).wait()
        pltpu.make_async_copy(dst_hbm.at[pl.ds(0, CHUNK_ROWS)], dst_v,
                              sem).wait()

    idx_bufs = [(src_a, dst_a, sem_ia), (src_b, dst_b, sem_ib)]
    for ch in range(NCHUNKS):
        src_v, dst_v, sem_i = idx_bufs[ch % 2]
        wait_idx(src_v, dst_v, sem_i)
        if ch == 0:
            init_d.wait()
            plsc.subcore_barrier()
        if ch < NCHUNKS - 1:
            nsrc, ndst, nsem = idx_bufs[(ch + 1) % 2]
            nbase = s * ROWS_PER_TILE + (ch + 1) * CHUNK_ROWS
            pltpu.async_copy(src_hbm.at[pl.ds(nbase, CHUNK_ROWS)], nsrc, nsem)
            pltpu.async_copy(dst_hbm.at[pl.ds(nbase, CHUNK_ROWS)], ndst, nsem)
        pltpu.async_copy(y_hbm.at[c].at[src_v.at[0]], rows_a, sem_a)
        pltpu.async_copy(y_hbm.at[c].at[src_v.at[1]], rows_b, sem_b)

        def pair_body(q, c2, src_v=src_v, dst_v=dst_v):
            j0 = 2 * q
            wait_gather(rows_a, sem_a)
            pltpu.sync_copy(rows_a, acc_sh.at[dst_v.at[j0]], add=True)

            @pl.when(q < npairs - 1)
            def _():
                pltpu.async_copy(y_hbm.at[c].at[src_v.at[j0 + 2]], rows_a,
                                 sem_a)
            wait_gather(rows_b, sem_b)
            pltpu.sync_copy(rows_b, acc_sh.at[dst_v.at[j0 + 1]], add=True)

            @pl.when(q < npairs - 1)
            def _():
                pltpu.async_copy(y_hbm.at[c].at[src_v.at[j0 + 3]], rows_b,
                                 sem_b)
            return c2
        lax.fori_loop(0, npairs, pair_body, 0)

    plsc.subcore_barrier()
    pltpu.sync_copy(acc_sh.at[rowsl], out_hbm.at[c].at[rowsl])


@functools.partial(
    pl.kernel,
    out_type=jax.ShapeDtypeStruct((NC, NACC, 128), jnp.float32),
    mesh=_sc_mesh,
    scratch_types=[
        pltpu.VMEM((CHUNK_ROWS, 128), jnp.int32),
        pltpu.VMEM((CHUNK_ROWS, 128), jnp.int32),
        pltpu.VMEM((CHUNK_ROWS, 128), jnp.int32),
        pltpu.VMEM((CHUNK_ROWS, 128), jnp.int32),
        pltpu.VMEM((128, 128), jnp.float32),
        pltpu.VMEM((128, 128), jnp.float32),
        pltpu.VMEM_SHARED((NACC, 128), jnp.float32),
        pltpu.SemaphoreType.DMA,
        pltpu.SemaphoreType.DMA,
        pltpu.SemaphoreType.DMA,
        pltpu.SemaphoreType.DMA,
        pltpu.SemaphoreType.DMA,
    ],
    compiler_params=pltpu.CompilerParams(needs_layout_passes=False),
)
def sc_scatter(y_hbm, src_hbm, dst_hbm, out_hbm, src_a, dst_a, src_b, dst_b,
               rows_a, rows_b, acc_sh, sem_a, sem_b, sem_ia, sem_ib, sem_init):
    _scatter_body(y_hbm, src_hbm, dst_hbm, out_hbm, src_a, dst_a, src_b,
                  dst_b, rows_a, rows_b, acc_sh, sem_a, sem_b, sem_ia, sem_ib,
                  sem_init)


def _mm1_body(x_ref, w_ref, d0_ref, d1_ref, o_ref):
    dis = lax.rsqrt(d0_ref[...] + d1_ref[...] + 1.0)
    xw = jnp.dot(x_ref[...], w_ref[...], preferred_element_type=jnp.float32)
    o_ref[0] = xw * dis


def tc_mm1(x, W1, deg0, deg1):
    return pl.pallas_call(
        _mm1_body,
        grid=(N // ROW_BLK, HID // 128),
        in_specs=[
            pl.BlockSpec((ROW_BLK, IN_F), lambda i, j: (i, 0)),
            pl.BlockSpec((IN_F, 128), lambda i, j: (0, j)),
            pl.BlockSpec((ROW_BLK, 1), lambda i, j: (i, 0)),
            pl.BlockSpec((ROW_BLK, 1), lambda i, j: (i, 0)),
        ],
        out_specs=pl.BlockSpec((1, ROW_BLK, 128), lambda i, j: (j, i, 0)),
        out_shape=jax.ShapeDtypeStruct((NC, NACC, 128), jnp.float32),
    )(x, W1, deg0, deg1)


def _mm2_body(a_ref, w_ref, d0_ref, d1_ref, b_ref, o_ref):
    dis = lax.rsqrt(d0_ref[...] + d1_ref[...] + 1.0)
    h = jnp.concatenate([a_ref[0], a_ref[1]], axis=1)
    h1 = jnp.maximum(h * dis + b_ref[...], 0.0)
    o_ref[0] = jnp.dot(h1, w_ref[...], preferred_element_type=jnp.float32) * dis


def tc_mm2(acc, W2, deg0, deg1, b1):
    return pl.pallas_call(
        _mm2_body,
        grid=(N // ROW_BLK, HID // 128),
        in_specs=[
            pl.BlockSpec((NC, ROW_BLK, 128), lambda i, j: (0, i, 0)),
            pl.BlockSpec((HID, 128), lambda i, j: (0, j)),
            pl.BlockSpec((ROW_BLK, 1), lambda i, j: (i, 0)),
            pl.BlockSpec((ROW_BLK, 1), lambda i, j: (i, 0)),
            pl.BlockSpec((1, HID), lambda i, j: (0, 0)),
        ],
        out_specs=pl.BlockSpec((1, ROW_BLK, 128), lambda i, j: (j, i, 0)),
        out_shape=jax.ShapeDtypeStruct((NC, NACC, 128), jnp.float32),
    )(acc, W2, deg0, deg1, b1)


def _heads_body(a_ref, d0_ref, d1_ref, b2_ref, dw1_ref, db1_ref, dw2_ref,
                db2_ref, cw1_ref, cb1_ref, cw2_ref, cb2_ref, ob_ref, ot_ref):
    dis = lax.rsqrt(d0_ref[...] + d1_ref[...] + 1.0)
    h = jnp.concatenate([a_ref[0], a_ref[1]], axis=1)
    h2 = jnp.maximum(h * dis + b2_ref[...], 0.0)
    t1 = jnp.maximum(
        jnp.dot(h2, dw1_ref[...], preferred_element_type=jnp.float32)
        + db1_ref[...], 0.0)
    ob_ref[...] = (jnp.dot(t1, dw2_ref[...], preferred_element_type=jnp.float32)
                   + db2_ref[...])
    t2 = jnp.maximum(
        jnp.dot(h2, cw1_ref[...], preferred_element_type=jnp.float32)
        + cb1_ref[...], 0.0)
    ot_ref[...] = (jnp.dot(t2, cw2_ref[...], preferred_element_type=jnp.float32)
                   + cb2_ref[...])


def tc_heads(acc, deg0, deg1, b2, dW1c, db1, dW2p, db2p, cW1c, cb1, cW2p, cb2p):
    return pl.pallas_call(
        _heads_body,
        grid=(N // ROW_BLK,),
        in_specs=[
            pl.BlockSpec((NC, ROW_BLK, 128), lambda i: (0, i, 0)),
            pl.BlockSpec((ROW_BLK, 1), lambda i: (i, 0)),
            pl.BlockSpec((ROW_BLK, 1), lambda i: (i, 0)),
            pl.BlockSpec((1, HID), lambda i: (0, 0)),
            pl.BlockSpec((HID, HID), lambda i: (0, 0)),
            pl.BlockSpec((1, HID), lambda i: (0, 0)),
            pl.BlockSpec((HID, 2), lambda i: (0, 0)),
            pl.BlockSpec((1, 2), lambda i: (0, 0)),
            pl.BlockSpec((HID, HID), lambda i: (0, 0)),
            pl.BlockSpec((1, HID), lambda i: (0, 0)),
            pl.BlockSpec((HID, 8), lambda i: (0, 0)),
            pl.BlockSpec((1, 8), lambda i: (0, 0)),
        ],
        out_specs=[
            pl.BlockSpec((ROW_BLK, 2), lambda i: (i, 0)),
            pl.BlockSpec((ROW_BLK, 8), lambda i: (i, 0)),
        ],
        out_shape=[
            jax.ShapeDtypeStruct((N, 2), jnp.float32),
            jax.ShapeDtypeStruct((N, 8), jnp.float32),
        ],
    )(acc, deg0, deg1, b2, dW1c, db1, dW2p, db2p, cW1c, cb1, cW2p, cb2p)


def kernel(x, edge_index, W1, b1, W2, b2, dW1, db1, dW2, db2, cW1, cb1, cW2,
           cb2):
    src = edge_index[0]
    dst = edge_index[1]
    pad = EPAD - E
    padv = jnp.arange(pad, dtype=jnp.int32)
    src2d = jnp.concatenate([src, padv % N]).reshape(EROWS, 128)
    dst2d = jnp.concatenate([dst, N + padv % (NACC - N)]).reshape(EROWS, 128)

    degp = sc_degree(dst2d)
    deg0 = degp[0][:N][:, None]
    deg1 = degp[1][:N][:, None]

    y1 = tc_mm1(x, W1, deg0, deg1)
    acc1 = sc_scatter(y1, src2d, dst2d)
    y2 = tc_mm2(acc1, W2, deg0, deg1, b1.reshape(1, HID))
    acc2 = sc_scatter(y2, src2d, dst2d)

    outb, outt = tc_heads(acc2, deg0, deg1, b2.reshape(1, HID), dW1[:HID],
                          db1.reshape(1, HID), dW2, db2.reshape(1, 2),
                          cW1[:HID], cb1.reshape(1, HID), cW2,
                          cb2.reshape(1, NUM_TYPES))
    return outb, outt


# speedup vs baseline: 1.1908x; 1.0027x over previous
import functools

import jax
import jax.numpy as jnp
from jax import lax
from jax.experimental import pallas as pl
from jax.experimental.pallas import tpu as pltpu
from jax.experimental.pallas import tpu_sc as plsc

N = 10000
E = 320000
IN_F = 128
HID = 256
NUM_TYPES = 8

NC = 2
NS = 16
L = 16

EROWS = 2560
EPAD = EROWS * 128
ROWS_PER_TILE = EROWS // NS
ROWS_PER_WORKER = EROWS // (NC * NS)
CHUNK_ROWS = 32

NACC = 10112
ACC_PER_TILE = NACC // NS
HROWS = 128

ROW_BLK = 2000

_sc_mesh = plsc.VectorSubcoreMesh(core_axis_name="c", subcore_axis_name="s")


HSIZE = HROWS * 128
HSEG = HSIZE // NS


def _deg_body(dst_hbm, out_hbm, idx_v, hist_v, red_v, outp_v, shared_h):
    c = lax.axis_index("c")
    s = lax.axis_index("s")
    w = c * NS + s
    pltpu.sync_copy(dst_hbm.at[pl.ds(w * ROWS_PER_WORKER, ROWS_PER_WORKER)], idx_v)
    zero16 = jnp.zeros((L,), jnp.float32)
    ones16 = jnp.ones((L,), jnp.float32)

    def zbody(i, carry):
        hist_v[pl.ds(i * L, L)] = zero16
        return carry
    lax.fori_loop(0, HSIZE // L, zbody, 0)

    def hbody(r, carry):
        for k in range(128 // L):
            idx16 = idx_v[r, pl.ds(k * L, L)]
            plsc.addupdate_scatter(hist_v, [idx16], ones16)
        return carry
    lax.fori_loop(0, ROWS_PER_WORKER, hbody, 0)

    pltpu.sync_copy(hist_v, shared_h.at[s])
    plsc.subcore_barrier()
    for t in range(NS):
        pltpu.sync_copy(shared_h.at[t].at[pl.ds(s * HSEG, HSEG)], red_v.at[t])
    for k in range(HSEG // L):
        acc = red_v[0, pl.ds(k * L, L)]
        for t in range(1, NS):
            acc = acc + red_v[t, pl.ds(k * L, L)]
        outp_v[pl.ds(k * L, L)] = acc
    pltpu.sync_copy(outp_v, out_hbm.at[c].at[pl.ds(s * HSEG, HSEG)])


@functools.partial(
    pl.kernel,
    out_type=jax.ShapeDtypeStruct((NC, HSIZE), jnp.float32),
    mesh=_sc_mesh,
    scratch_types=[
        pltpu.VMEM((ROWS_PER_WORKER, 128), jnp.int32),
        pltpu.VMEM((HSIZE,), jnp.float32),
        pltpu.VMEM((NS, HSEG), jnp.float32),
        pltpu.VMEM((HSEG,), jnp.float32),
        pltpu.VMEM_SHARED((NS, HSIZE), jnp.float32),
    ],
    compiler_params=pltpu.CompilerParams(needs_layout_passes=False),
)
def sc_degree(dst_hbm, out_hbm, idx_v, hist_v, red_v, outp_v, shared_h):
    _deg_body(dst_hbm, out_hbm, idx_v, hist_v, red_v, outp_v, shared_h)


NCHUNKS = ROWS_PER_TILE // CHUNK_ROWS


def _scatter_body(y_hbm, src_hbm, dst_hbm, out_hbm, src_a, dst_a, src_b,
                  dst_b, rows_a, rows_b, acc_sh, sem_a, sem_b, sem_ia, sem_ib,
                  sem_init):
    c = lax.axis_index("c")
    s = lax.axis_index("s")
    rowsl = pl.ds(s * ACC_PER_TILE, ACC_PER_TILE)
    npairs = CHUNK_ROWS // 2

    init_d = pltpu.async_copy(y_hbm.at[c].at[rowsl], acc_sh.at[rowsl],
                              sem_init)
    base0 = s * ROWS_PER_TILE
    pltpu.async_copy(src_hbm.at[pl.ds(base0, CHUNK_ROWS)], src_a, sem_ia)
    pltpu.async_copy(dst_hbm.at[pl.ds(base0, CHUNK_ROWS)], dst_a, sem_ia)

    def wait_gather(rows_v, sem):
        pltpu.make_async_copy(y_hbm.at[c].at[pl.ds(0, 128)], rows_v, sem).wait()

    def wait_idx(src_v, dst_v, sem):
        pltpu.make_async_copy(src_hbm.at[pl.ds(0, CHUNK_ROWS)], src_v,
                              sem).wait()
        pltpu.make_async_copy(dst_hbm.at[pl.ds(0, CHUNK_ROWS)], dst_v,
                              sem).wait()

    idx_bufs = [(src_a, dst_a, sem_ia), (src_b, dst_b, sem_ib)]
    for ch in range(NCHUNKS):
        src_v, dst_v, sem_i = idx_bufs[ch % 2]
        wait_idx(src_v, dst_v, sem_i)
        if ch == 0:
            init_d.wait()
            plsc.subcore_barrier()
        if ch < NCHUNKS - 1:
            nsrc, ndst, nsem = idx_bufs[(ch + 1) % 2]
            nbase = s * ROWS_PER_TILE + (ch + 1) * CHUNK_ROWS
            pltpu.async_copy(src_hbm.at[pl.ds(nbase, CHUNK_ROWS)], nsrc, nsem)
            pltpu.async_copy(dst_hbm.at[pl.ds(nbase, CHUNK_ROWS)], ndst, nsem)
        pltpu.async_copy(y_hbm.at[c].at[src_v.at[0]], rows_a, sem_a)
        pltpu.async_copy(y_hbm.at[c].at[src_v.at[1]], rows_b, sem_b)

        def pair_body(q, c2, src_v=src_v, dst_v=dst_v):
            j0 = 2 * q
            wait_gather(rows_a, sem_a)
            pltpu.sync_copy(rows_a, acc_sh.at[dst_v.at[j0]], add=True)

            @pl.when(q < npairs - 1)
            def _():
                pltpu.async_copy(y_hbm.at[c].at[src_v.at[j0 + 2]], rows_a,
                                 sem_a)
            wait_gather(rows_b, sem_b)
            pltpu.sync_copy(rows_b, acc_sh.at[dst_v.at[j0 + 1]], add=True)

            @pl.when(q < npairs - 1)
            def _():
                pltpu.async_copy(y_hbm.at[c].at[src_v.at[j0 + 3]], rows_b,
                                 sem_b)
            return c2
        lax.fori_loop(0, npairs, pair_body, 0)

    plsc.subcore_barrier()
    pltpu.sync_copy(acc_sh.at[rowsl], out_hbm.at[c].at[rowsl])


@functools.partial(
    pl.kernel,
    out_type=jax.ShapeDtypeStruct((NC, NACC, 128), jnp.float32),
    mesh=_sc_mesh,
    scratch_types=[
        pltpu.VMEM((CHUNK_ROWS, 128), jnp.int32),
        pltpu.VMEM((CHUNK_ROWS, 128), jnp.int32),
        pltpu.VMEM((CHUNK_ROWS, 128), jnp.int32),
        pltpu.VMEM((CHUNK_ROWS, 128), jnp.int32),
        pltpu.VMEM((128, 128), jnp.float32),
        pltpu.VMEM((128, 128), jnp.float32),
        pltpu.VMEM_SHARED((NACC, 128), jnp.float32),
        pltpu.SemaphoreType.DMA,
        pltpu.SemaphoreType.DMA,
        pltpu.SemaphoreType.DMA,
        pltpu.SemaphoreType.DMA,
        pltpu.SemaphoreType.DMA,
    ],
    compiler_params=pltpu.CompilerParams(needs_layout_passes=False),
)
def sc_scatter(y_hbm, src_hbm, dst_hbm, out_hbm, src_a, dst_a, src_b, dst_b,
               rows_a, rows_b, acc_sh, sem_a, sem_b, sem_ia, sem_ib, sem_init):
    _scatter_body(y_hbm, src_hbm, dst_hbm, out_hbm, src_a, dst_a, src_b,
                  dst_b, rows_a, rows_b, acc_sh, sem_a, sem_b, sem_ia, sem_ib,
                  sem_init)


def _mm1_body(x_ref, w_ref, d0_ref, d1_ref, o_ref):
    dis = lax.rsqrt(d0_ref[...] + d1_ref[...] + 1.0)
    xw = jnp.dot(x_ref[...], w_ref[...], preferred_element_type=jnp.float32)
    o_ref[0] = xw * dis


def tc_mm1(x, W1, deg0, deg1):
    return pl.pallas_call(
        _mm1_body,
        grid=(N // ROW_BLK, HID // 128),
        in_specs=[
            pl.BlockSpec((ROW_BLK, IN_F), lambda i, j: (i, 0)),
            pl.BlockSpec((IN_F, 128), lambda i, j: (0, j)),
            pl.BlockSpec((ROW_BLK, 1), lambda i, j: (i, 0)),
            pl.BlockSpec((ROW_BLK, 1), lambda i, j: (i, 0)),
        ],
        out_specs=pl.BlockSpec((1, ROW_BLK, 128), lambda i, j: (j, i, 0)),
        out_shape=jax.ShapeDtypeStruct((NC, NACC, 128), jnp.float32),
    )(x, W1, deg0, deg1)


def _mm2_body(a_ref, w_ref, d0_ref, d1_ref, b_ref, o_ref):
    dis = lax.rsqrt(d0_ref[...] + d1_ref[...] + 1.0)
    h = jnp.concatenate([a_ref[0], a_ref[1]], axis=1)
    h1 = jnp.maximum(h * dis + b_ref[...], 0.0)
    o_ref[0] = jnp.dot(h1, w_ref[...], preferred_element_type=jnp.float32) * dis


def tc_mm2(acc, W2, deg0, deg1, b1):
    return pl.pallas_call(
        _mm2_body,
        grid=(N // ROW_BLK, HID // 128),
        in_specs=[
            pl.BlockSpec((NC, ROW_BLK, 128), lambda i, j: (0, i, 0)),
            pl.BlockSpec((HID, 128), lambda i, j: (0, j)),
            pl.BlockSpec((ROW_BLK, 1), lambda i, j: (i, 0)),
            pl.BlockSpec((ROW_BLK, 1), lambda i, j: (i, 0)),
            pl.BlockSpec((1, HID), lambda i, j: (0, 0)),
        ],
        out_specs=pl.BlockSpec((1, ROW_BLK, 128), lambda i, j: (j, i, 0)),
        out_shape=jax.ShapeDtypeStruct((NC, NACC, 128), jnp.float32),
    )(acc, W2, deg0, deg1, b1)


def _heads_body(a_ref, d0_ref, d1_ref, b2_ref, dw1_ref, db1_ref, dw2_ref,
                db2_ref, cw1_ref, cb1_ref, cw2_ref, cb2_ref, ob_ref, ot_ref):
    dis = lax.rsqrt(d0_ref[...] + d1_ref[...] + 1.0)
    h = jnp.concatenate([a_ref[0], a_ref[1]], axis=1)
    h2 = jnp.maximum(h * dis + b2_ref[...], 0.0)
    t1 = jnp.maximum(
        jnp.dot(h2, dw1_ref[...], preferred_element_type=jnp.float32)
        + db1_ref[...], 0.0)
    ob_ref[...] = (jnp.dot(t1, dw2_ref[...], preferred_element_type=jnp.float32)
                   + db2_ref[...])
    t2 = jnp.maximum(
        jnp.dot(h2, cw1_ref[...], preferred_element_type=jnp.float32)
        + cb1_ref[...], 0.0)
    ot_ref[...] = (jnp.dot(t2, cw2_ref[...], preferred_element_type=jnp.float32)
                   + cb2_ref[...])


def tc_heads(acc, deg0, deg1, b2, dW1c, db1, dW2p, db2p, cW1c, cb1, cW2p, cb2p):
    return pl.pallas_call(
        _heads_body,
        grid=(N // ROW_BLK,),
        in_specs=[
            pl.BlockSpec((NC, ROW_BLK, 128), lambda i: (0, i, 0)),
            pl.BlockSpec((ROW_BLK, 1), lambda i: (i, 0)),
            pl.BlockSpec((ROW_BLK, 1), lambda i: (i, 0)),
            pl.BlockSpec((1, HID), lambda i: (0, 0)),
            pl.BlockSpec((HID, HID), lambda i: (0, 0)),
            pl.BlockSpec((1, HID), lambda i: (0, 0)),
            pl.BlockSpec((HID, 2), lambda i: (0, 0)),
            pl.BlockSpec((1, 2), lambda i: (0, 0)),
            pl.BlockSpec((HID, HID), lambda i: (0, 0)),
            pl.BlockSpec((1, HID), lambda i: (0, 0)),
            pl.BlockSpec((HID, 8), lambda i: (0, 0)),
            pl.BlockSpec((1, 8), lambda i: (0, 0)),
        ],
        out_specs=[
            pl.BlockSpec((ROW_BLK, 2), lambda i: (i, 0)),
            pl.BlockSpec((ROW_BLK, 8), lambda i: (i, 0)),
        ],
        out_shape=[
            jax.ShapeDtypeStruct((N, 2), jnp.float32),
            jax.ShapeDtypeStruct((N, 8), jnp.float32),
        ],
    )(acc, deg0, deg1, b2, dW1c, db1, dW2p, db2p, cW1c, cb1, cW2p, cb2p)


def kernel(x, edge_index, W1, b1, W2, b2, dW1, db1, dW2, db2, cW1, cb1, cW2,
           cb2):
    src = edge_index[0]
    dst = edge_index[1]
    pad = EPAD - E
    padv = jnp.arange(pad, dtype=jnp.int32)
    src2d = jnp.concatenate([src, padv % N]).reshape(EROWS, 128)
    dst2d = jnp.concatenate([dst, N + padv % (NACC - N)]).reshape(EROWS, 128)

    degp = sc_degree(dst2d)
    deg0 = degp[0][:N][:, None]
    deg1 = degp[1][:N][:, None]

    y1 = tc_mm1(x, W1, deg0, deg1)
    acc1 = sc_scatter(y1, src2d, dst2d)
    y2 = tc_mm2(acc1, W2, deg0, deg1, b1.reshape(1, HID))
    acc2 = sc_scatter(y2, src2d, dst2d)

    outb, outt = tc_heads(acc2, deg0, deg1, b2.reshape(1, HID), dW1[:HID],
                          db1.reshape(1, HID), dW2, db2.reshape(1, 2),
                          cW1[:HID], cb1.reshape(1, HID), cW2,
                          cb2.reshape(1, NUM_TYPES))
    return outb, outt
